# baseline (speedup 1.0000x reference)
import functools

import jax
import jax.numpy as jnp
from jax import lax
from jax.experimental import pallas as pl
from jax.experimental.pallas import tpu as pltpu
from jax.experimental.pallas import tpu_sc as plsc

BATCH = 16384
EMBED_DIM = 32
NROWS = 1000000
PACK = 4
PROWS = NROWS // PACK
L = 16
NC = 2
NS = 16
NW = NC * NS
BPW = BATCH // NW
CHUNK = 128
NCHUNK = BPW // CHUNK
NBUF = 2


def _mesh():
    return plsc.VectorSubcoreMesh(core_axis_name="c", subcore_axis_name="s")


@functools.partial(
    pl.kernel,
    out_type=jax.ShapeDtypeStruct((BATCH,), jnp.float32),
    mesh=_mesh(),
    scratch_types=[
        pltpu.VMEM((BPW,), jnp.int32),
        pltpu.VMEM((BPW,), jnp.int32),
        pltpu.VMEM((BPW,), jnp.int32),
        pltpu.VMEM((BPW,), jnp.int32),
        pltpu.VMEM((BPW + L,), jnp.int32),
        pltpu.VMEM((BPW + L,), jnp.int32),
        pltpu.VMEM((NBUF, CHUNK, 128), jnp.float32),
        pltpu.VMEM((NBUF, CHUNK, 128), jnp.float32),
        pltpu.VMEM((4 * L,), jnp.float32),
        pltpu.VMEM((L,), jnp.float32),
        pltpu.VMEM((BPW,), jnp.float32),
        pltpu.SemaphoreType.DMA,
        pltpu.SemaphoreType.DMA,
    ],
    compiler_params=pltpu.CompilerParams(needs_layout_passes=False),
)
def _sc_kernel(users_hbm, movies_hbm, ut_hbm, mt_hbm, w_hbm, b16_hbm, out_hbm,
               uidx_v, midx_v, ush_v, msh_v, uoff_v, moff_v,
               urows_v, mrows_v, w_v, b16_v, acc_v, usem, msem):
    wid = lax.axis_index("s") * NC + lax.axis_index("c")
    base = wid * BPW

    pltpu.sync_copy(users_hbm.at[pl.ds(base, BPW)], uidx_v)
    pltpu.sync_copy(movies_hbm.at[pl.ds(base, BPW)], midx_v)
    pltpu.sync_copy(w_hbm, w_v)
    pltpu.sync_copy(b16_hbm, b16_v)

    def prep(k, _):
        sl = pl.ds(k * L, L)
        for idx_v, sh_v, off_v in ((uidx_v, ush_v, uoff_v),
                                   (midx_v, msh_v, moff_v)):
            v = idx_v[sl]
            z = jnp.zeros((L,), jnp.int32)
            row = v
            off = z
            for q in range(1, PACK):
                ge = v >= q * PROWS
                row = row - jnp.where(ge, PROWS, 0)
                off = off + jnp.where(ge, EMBED_DIM, 0)
            sh_v[sl] = row
            off_v[sl] = off
        return 0

    lax.fori_loop(0, BPW // L, prep, 0, unroll=4)

    def fire(j):
        slot = j % NBUF
        sl = pl.ds(j * CHUNK, CHUNK)
        uc = pltpu.async_copy(ut_hbm.at[ush_v.at[sl]], urows_v.at[slot], usem)
        mc = pltpu.async_copy(mt_hbm.at[msh_v.at[sl]], mrows_v.at[slot], msem)
        return uc, mc

    copies = [fire(0), fire(1)]

    w_u0 = w_v[pl.ds(0, L)]
    w_u1 = w_v[pl.ds(L, L)]
    w_m0 = w_v[pl.ds(2 * L, L)]
    w_m1 = w_v[pl.ds(3 * L, L)]
    b16 = b16_v[...]
    msk15 = lax.iota(jnp.int32, L) == (L - 1)

    for j in range(NCHUNK):
        slot = j % NBUF
        uc, mc = copies[j]
        uc.wait()
        mc.wait()

        def body(i, _, j=j, slot=slot):
            g = j * CHUNK + i
            off_u = uoff_v[pl.ds(g, L)][0]
            off_m = moff_v[pl.ds(g, L)][0]
            u0 = urows_v[slot, i, pl.ds(off_u, L)]
            u1 = urows_v[slot, i, pl.ds(off_u + L, L)]
            m0 = mrows_v[slot, i, pl.ds(off_m, L)]
            m1 = mrows_v[slot, i, pl.ds(off_m + L, L)]
            acc = u0 * w_u0 + u1 * w_u1 + m0 * w_m0 + m1 * w_m1 + b16
            tot = plsc.cumsum(acc)
            idxv = jnp.full((L,), g, dtype=jnp.int32)
            plsc.store_scatter(acc_v, [idxv], tot, mask=msk15)
            return 0

        lax.fori_loop(0, CHUNK, body, 0, unroll=4)
        if j + NBUF < NCHUNK:
            copies.append(fire(j + NBUF))

    pltpu.sync_copy(acc_v, out_hbm.at[pl.ds(base, BPW)])


def _lane_pack(table):
    return jnp.concatenate(
        [table[q * PROWS:(q + 1) * PROWS] for q in range(PACK)], axis=1)


def kernel(users, movies, user_table, movie_table, W, b):
    ut = _lane_pack(user_table)
    mt = _lane_pack(movie_table)
    w_flat = W.reshape(4 * L).astype(jnp.float32)
    b16 = jnp.full((L,), b[0] / L, dtype=jnp.float32)
    out = _sc_kernel(users.astype(jnp.int32), movies.astype(jnp.int32),
                     ut, mt, w_flat, b16)
    return out.reshape(BATCH, 1)


# speedup vs baseline: 1.9226x; 1.9226x over previous
import functools

import jax
import jax.numpy as jnp
from jax import lax
from jax.experimental import pallas as pl
from jax.experimental.pallas import tpu as pltpu
from jax.experimental.pallas import tpu_sc as plsc

BATCH = 16384
EMBED_DIM = 32
NROWS = 1000000
L = 16
NC = 2
NS = 16
NW = NC * NS
BPW = BATCH // NW
CHUNK = 128
NCHUNK = BPW // CHUNK
NBUF = 2


def _mesh():
    return plsc.VectorSubcoreMesh(core_axis_name="c", subcore_axis_name="s")


@functools.partial(
    pl.kernel,
    out_type=jax.ShapeDtypeStruct((BATCH,), jnp.float32),
    mesh=_mesh(),
    scratch_types=[
        pltpu.VMEM((BPW + L,), jnp.int32),
        pltpu.VMEM((BPW + L,), jnp.int32),
        pltpu.VMEM((NBUF, CHUNK, EMBED_DIM), jnp.float32),
        pltpu.VMEM((NBUF, CHUNK, EMBED_DIM), jnp.float32),
        pltpu.VMEM((4 * L,), jnp.float32),
        pltpu.VMEM((L,), jnp.float32),
        pltpu.VMEM((BPW,), jnp.float32),
        pltpu.SemaphoreType.DMA,
        pltpu.SemaphoreType.DMA,
    ],
    compiler_params=pltpu.CompilerParams(needs_layout_passes=False),
)
def _sc_kernel(users_hbm, movies_hbm, ut_hbm, mt_hbm, w_hbm, b16_hbm, out_hbm,
               uidx_v, midx_v, urows_v, mrows_v, w_v, b16_v, acc_v,
               usem, msem):
    wid = lax.axis_index("s") * NC + lax.axis_index("c")
    base = wid * BPW

    pltpu.sync_copy(users_hbm.at[pl.ds(base, BPW)], uidx_v.at[pl.ds(0, BPW)])
    pltpu.sync_copy(movies_hbm.at[pl.ds(base, BPW)], midx_v.at[pl.ds(0, BPW)])
    pltpu.sync_copy(w_hbm, w_v)
    pltpu.sync_copy(b16_hbm, b16_v)

    def fire(j):
        slot = j % NBUF

        def en(i, _, j=j, slot=slot):
            g = j * CHUNK + i
            ru = uidx_v[pl.ds(g, L)][0]
            rm = midx_v[pl.ds(g, L)][0]
            pltpu.async_copy(ut_hbm.at[ru], urows_v.at[slot, i], usem)
            pltpu.async_copy(mt_hbm.at[rm], mrows_v.at[slot, i], msem)
            return 0

        lax.fori_loop(0, CHUNK, en, 0, unroll=2)

    def drain(slot):
        def wt(i, _, slot=slot):
            pltpu.make_async_copy(ut_hbm.at[0], urows_v.at[slot, i], usem).wait()
            pltpu.make_async_copy(mt_hbm.at[0], mrows_v.at[slot, i], msem).wait()
            return 0

        lax.fori_loop(0, CHUNK, wt, 0, unroll=2)

    fire(0)
    if NCHUNK > 1:
        fire(1)

    w_u0 = w_v[pl.ds(0, L)]
    w_u1 = w_v[pl.ds(L, L)]
    w_m0 = w_v[pl.ds(2 * L, L)]
    w_m1 = w_v[pl.ds(3 * L, L)]
    b16 = b16_v[...]
    msk15 = lax.iota(jnp.int32, L) == (L - 1)

    for j in range(NCHUNK):
        slot = j % NBUF
        drain(slot)

        def body(i, _, j=j, slot=slot):
            g = j * CHUNK + i
            u0 = urows_v[slot, i, pl.ds(0, L)]
            u1 = urows_v[slot, i, pl.ds(L, L)]
            m0 = mrows_v[slot, i, pl.ds(0, L)]
            m1 = mrows_v[slot, i, pl.ds(L, L)]
            acc = u0 * w_u0 + u1 * w_u1 + m0 * w_m0 + m1 * w_m1 + b16
            tot = plsc.cumsum(acc)
            idxv = jnp.full((L,), g, dtype=jnp.int32)
            plsc.store_scatter(acc_v, [idxv], tot, mask=msk15)
            return 0

        lax.fori_loop(0, CHUNK, body, 0, unroll=4)
        if j + NBUF < NCHUNK:
            fire(j + NBUF)

    pltpu.sync_copy(acc_v, out_hbm.at[pl.ds(base, BPW)])


def kernel(users, movies, user_table, movie_table, W, b):
    w_flat = W.reshape(4 * L).astype(jnp.float32)
    b16 = jnp.full((L,), b[0] / L, dtype=jnp.float32)
    out = _sc_kernel(users.astype(jnp.int32), movies.astype(jnp.int32),
                     user_table, movie_table, w_flat, b16)
    return out.reshape(BATCH, 1)
